# Initial kernel scaffold; baseline (speedup 1.0000x reference)
import jax
import jax.numpy as jnp
from jax.experimental import pallas as pl


def kernel(logits, labels):
    raise NotImplementedError("write your pallas kernel here")


# speedup vs baseline: 16.3291x; 16.3291x over previous
import jax
import jax.numpy as jnp
from jax.experimental import pallas as pl
from jax.experimental.pallas import tpu as pltpu

_IGNORE = 255
_B, _C, _H, _W = 8, 19, 512, 512
_HB = 64
_N = _B * _H * _W
_K = int(0.2 * _N)
_GB, _GH = _B, _H // _HB


def _ce_topk_kernel(logits_ref, labels_ref, out_ref, bits_ref):
    b = pl.program_id(0)
    h = pl.program_id(1)

    x = logits_ref[0]
    lab = labels_ref[0]

    m = jnp.max(x, axis=0)
    s = jnp.sum(jnp.exp(x - m[None, :, :]), axis=0)
    cls = jax.lax.broadcasted_iota(jnp.int32, (_C, _HB, _W), 0)
    picked = jnp.sum(jnp.where(cls == lab[None, :, :], x, 0.0), axis=0)
    nll = (m - picked) + jnp.log(s)
    loss = jnp.where(lab != _IGNORE, nll, 0.0)

    bits_ref[pl.ds(h * _HB, _HB), pl.ds(b * _W, _W)] = pltpu.bitcast(
        loss, jnp.int32)

    @pl.when((b == _GB - 1) & (h == _GH - 1))
    def _select():
        bits = bits_ref[:, :]
        kf = jnp.float32(_K)

        def body(_, carry):
            lo, hi = carry
            mid = lo + (hi - lo) // 2
            cnt = jnp.sum((bits > mid).astype(jnp.float32))
            active = lo < hi
            below = cnt < kf
            new_hi = jnp.where(active & below, mid, hi)
            new_lo = jnp.where(active & (~below), mid + 1, lo)
            return new_lo, new_hi

        lo, _hi = jax.lax.fori_loop(
            0, 31, body, (jnp.int32(0), jnp.int32(0x7F800000)))

        vals = pltpu.bitcast(bits, jnp.float32)
        gt = bits > lo
        eq = bits == lo
        cnt_gt = jnp.sum(gt.astype(jnp.float32))
        sum_gt = jnp.sum(jnp.where(gt, vals, 0.0))
        cnt_eq = jnp.sum(eq.astype(jnp.float32))
        sum_eq = jnp.sum(jnp.where(eq, vals, 0.0))
        kth = sum_eq / cnt_eq
        out_ref[0, 0] = (sum_gt + (kf - cnt_gt) * kth) / kf


def kernel(logits, labels):
    out = pl.pallas_call(
        _ce_topk_kernel,
        grid=(_GB, _GH),
        in_specs=[
            pl.BlockSpec((1, _C, _HB, _W), lambda b, h: (b, 0, h, 0)),
            pl.BlockSpec((1, _HB, _W), lambda b, h: (b, h, 0)),
        ],
        out_specs=pl.BlockSpec(memory_space=pltpu.SMEM),
        out_shape=jax.ShapeDtypeStruct((1, 1), jnp.float32),
        scratch_shapes=[pltpu.VMEM((_H, _B * _W), jnp.int32)],
    )(logits, labels)
    return out[0, 0]


# speedup vs baseline: 26.5888x; 1.6283x over previous
import jax
import jax.numpy as jnp
from jax.experimental import pallas as pl
from jax.experimental.pallas import tpu as pltpu

_IGNORE = 255
_B, _C, _H, _W = 8, 19, 512, 512
_HB = 64
_N = _B * _H * _W
_K = int(0.2 * _N)
_GB, _GH = _B, _H // _HB
_SROWS, _SCOLS = _H, _B * _W


def _ce_topk_kernel(logits_ref, labels_ref, out_ref, val_ref, p16_ref):
    b = pl.program_id(0)
    h = pl.program_id(1)

    x = logits_ref[0]
    lab = labels_ref[0]

    m = jnp.max(x, axis=0)
    s = jnp.sum(jnp.exp(x - m[None, :, :]), axis=0)
    cls = jax.lax.broadcasted_iota(jnp.int32, (_C, _HB, _W), 0)
    picked = jnp.sum(jnp.where(cls == lab[None, :, :], x, 0.0), axis=0)
    nll = (m - picked) + jnp.log(s)
    loss = jnp.where(lab != _IGNORE, nll, 0.0)

    val_ref[pl.ds(h * _HB, _HB), pl.ds(b * _W, _W)] = loss
    p16_ref[pl.ds(h * _HB, _HB), pl.ds(b * _W, _W)] = pltpu.bitcast(
        loss.astype(jnp.bfloat16), jnp.int16)

    @pl.when((b == _GB - 1) & (h == _GH - 1))
    def _select():
        kf = jnp.float32(_K)
        one16 = jnp.int16(1)

        def count_gt(mid):
            mid16 = mid.astype(jnp.int16)
            acc = jnp.zeros((16, _SCOLS), jnp.int16)
            for j in range(_SROWS // 16):
                blk = p16_ref[pl.ds(j * 16, 16), :]
                acc = acc + jnp.where(blk > mid16, one16, jnp.int16(0))
            return jnp.sum(acc.astype(jnp.float32))

        def body(_, carry):
            lo, hi = carry
            mid = lo + (hi - lo) // 2
            cnt = count_gt(mid)
            active = lo < hi
            below = cnt < kf
            new_hi = jnp.where(active & below, mid, hi)
            new_lo = jnp.where(active & (~below), mid + 1, lo)
            return new_lo, new_hi

        lo, _hi = jax.lax.fori_loop(
            0, 15, body, (jnp.int32(0), jnp.int32(0x7F80)))
        tf = lo.astype(jnp.float32)

        zf = jnp.zeros((8, _SCOLS), jnp.float32)
        s_gt, c_gt, s_eq, c_eq = zf, zf, zf, zf
        for j in range(_SROWS // 8):
            p = p16_ref[pl.ds(j * 8, 8), :].astype(jnp.float32)
            v = val_ref[pl.ds(j * 8, 8), :]
            gt = p > tf
            eq = p == tf
            s_gt = s_gt + jnp.where(gt, v, 0.0)
            c_gt = c_gt + jnp.where(gt, 1.0, 0.0)
            s_eq = s_eq + jnp.where(eq, v, 0.0)
            c_eq = c_eq + jnp.where(eq, 1.0, 0.0)
        sum_gt = jnp.sum(s_gt)
        cnt_gt = jnp.sum(c_gt)
        sum_eq = jnp.sum(s_eq)
        cnt_eq = jnp.maximum(jnp.sum(c_eq), 1.0)
        kth = sum_eq / cnt_eq
        out_ref[0, 0] = (sum_gt + (kf - cnt_gt) * kth) / kf


def kernel(logits, labels):
    out = pl.pallas_call(
        _ce_topk_kernel,
        grid=(_GB, _GH),
        in_specs=[
            pl.BlockSpec((1, _C, _HB, _W), lambda b, h: (b, 0, h, 0)),
            pl.BlockSpec((1, _HB, _W), lambda b, h: (b, h, 0)),
        ],
        out_specs=pl.BlockSpec(memory_space=pltpu.SMEM),
        out_shape=jax.ShapeDtypeStruct((1, 1), jnp.float32),
        scratch_shapes=[
            pltpu.VMEM((_SROWS, _SCOLS), jnp.float32),
            pltpu.VMEM((_SROWS, _SCOLS), jnp.int16),
        ],
    )(logits, labels)
    return out[0, 0]


# speedup vs baseline: 32.7277x; 1.2309x over previous
import jax
import jax.numpy as jnp
from jax.experimental import pallas as pl
from jax.experimental.pallas import tpu as pltpu

_IGNORE = 255
_B, _C, _H, _W = 8, 19, 512, 512
_HB = 128
_N = _B * _H * _W
_K = int(0.2 * _N)
_GB, _GH = _B, _H // _HB
_SROWS, _SCOLS = _H, _B * _W


def _ce_topk_kernel(logits_ref, labels_ref, out_ref, val_ref, p16_ref):
    b = pl.program_id(0)
    h = pl.program_id(1)

    x = logits_ref[0]
    lab = labels_ref[0]

    m = jnp.max(x, axis=0)
    s = jnp.sum(jnp.exp(x - m[None, :, :]), axis=0)
    cls = jax.lax.broadcasted_iota(jnp.int32, (_C, _HB, _W), 0)
    picked = jnp.sum(jnp.where(cls == lab[None, :, :], x, 0.0), axis=0)
    nll = (m - picked) + jnp.log(s)
    loss = jnp.where(lab != _IGNORE, nll, 0.0)

    val_ref[pl.ds(h * _HB, _HB), pl.ds(b * _W, _W)] = loss
    p16_ref[pl.ds(h * _HB, _HB), pl.ds(b * _W, _W)] = pltpu.bitcast(
        loss.astype(jnp.bfloat16), jnp.int16)

    @pl.when((b == _GB - 1) & (h == _GH - 1))
    def _select():
        kf = jnp.float32(_K)
        one16 = jnp.int16(1)

        def count_gt(mid):
            mid16 = mid.astype(jnp.int16)
            acc = jnp.zeros((16, _SCOLS), jnp.int16)
            for j in range(_SROWS // 16):
                blk = p16_ref[pl.ds(j * 16, 16), :]
                acc = acc + jnp.where(blk > mid16, one16, jnp.int16(0))
            return jnp.sum(acc.astype(jnp.float32))

        def body(_, carry):
            lo, hi = carry
            mid = lo + (hi - lo) // 2
            cnt = count_gt(mid)
            active = lo < hi
            below = cnt < kf
            new_hi = jnp.where(active & below, mid, hi)
            new_lo = jnp.where(active & (~below), mid + 1, lo)
            return new_lo, new_hi

        lo, _hi = jax.lax.fori_loop(
            0, 15, body, (jnp.int32(0), jnp.int32(0x7F80)))
        tf = lo.astype(jnp.float32)

        zf = jnp.zeros((8, _SCOLS), jnp.float32)
        s_gt, c_gt, s_eq, c_eq = zf, zf, zf, zf
        for j in range(_SROWS // 8):
            p = p16_ref[pl.ds(j * 8, 8), :].astype(jnp.float32)
            v = val_ref[pl.ds(j * 8, 8), :]
            gt = p > tf
            eq = p == tf
            s_gt = s_gt + jnp.where(gt, v, 0.0)
            c_gt = c_gt + jnp.where(gt, 1.0, 0.0)
            s_eq = s_eq + jnp.where(eq, v, 0.0)
            c_eq = c_eq + jnp.where(eq, 1.0, 0.0)
        sum_gt = jnp.sum(s_gt)
        cnt_gt = jnp.sum(c_gt)
        sum_eq = jnp.sum(s_eq)
        cnt_eq = jnp.maximum(jnp.sum(c_eq), 1.0)
        kth = sum_eq / cnt_eq
        out_ref[0, 0] = (sum_gt + (kf - cnt_gt) * kth) / kf


def kernel(logits, labels):
    out = pl.pallas_call(
        _ce_topk_kernel,
        grid=(_GB, _GH),
        in_specs=[
            pl.BlockSpec((1, _C, _HB, _W), lambda b, h: (b, 0, h, 0)),
            pl.BlockSpec((1, _HB, _W), lambda b, h: (b, h, 0)),
        ],
        out_specs=pl.BlockSpec(memory_space=pltpu.SMEM),
        out_shape=jax.ShapeDtypeStruct((1, 1), jnp.float32),
        scratch_shapes=[
            pltpu.VMEM((_SROWS, _SCOLS), jnp.float32),
            pltpu.VMEM((_SROWS, _SCOLS), jnp.int16),
        ],
    )(logits, labels)
    return out[0, 0]


# speedup vs baseline: 36.2103x; 1.1064x over previous
import jax
import jax.numpy as jnp
from jax.experimental import pallas as pl
from jax.experimental.pallas import tpu as pltpu

_IGNORE = 255
_B, _C, _H, _W = 8, 19, 512, 512
_HB = 256
_N = _B * _H * _W
_K = int(0.2 * _N)
_GB, _GH = _B, _H // _HB
_SROWS, _SCOLS = _H, _B * _W


def _ce_topk_kernel(logits_ref, labels_ref, out_ref, val_ref, p16_ref):
    b = pl.program_id(0)
    h = pl.program_id(1)

    x = logits_ref[0]
    lab = labels_ref[0]

    m = jnp.max(x, axis=0)
    s = jnp.sum(jnp.exp(x - m[None, :, :]), axis=0)
    cls = jax.lax.broadcasted_iota(jnp.int32, (_C, _HB, _W), 0)
    picked = jnp.sum(jnp.where(cls == lab[None, :, :], x, 0.0), axis=0)
    nll = (m - picked) + jnp.log(s)
    loss = jnp.where(lab != _IGNORE, nll, 0.0)

    val_ref[pl.ds(h * _HB, _HB), pl.ds(b * _W, _W)] = loss
    p16_ref[pl.ds(h * _HB, _HB), pl.ds(b * _W, _W)] = pltpu.bitcast(
        loss.astype(jnp.bfloat16), jnp.int16)

    @pl.when((b == _GB - 1) & (h == _GH - 1))
    def _select():
        kf = jnp.float32(_K)
        one16 = jnp.int16(1)

        def count_gt(mid):
            mid16 = mid.astype(jnp.int16)
            acc = jnp.zeros((16, _SCOLS), jnp.int16)
            for j in range(_SROWS // 16):
                blk = p16_ref[pl.ds(j * 16, 16), :]
                acc = acc + jnp.where(blk > mid16, one16, jnp.int16(0))
            return jnp.sum(acc.astype(jnp.float32))

        def body(_, carry):
            lo, hi = carry
            mid = lo + (hi - lo) // 2
            cnt = count_gt(mid)
            active = lo < hi
            below = cnt < kf
            new_hi = jnp.where(active & below, mid, hi)
            new_lo = jnp.where(active & (~below), mid + 1, lo)
            return new_lo, new_hi

        lo, _hi = jax.lax.fori_loop(
            0, 15, body, (jnp.int32(0), jnp.int32(0x7F80)))
        tf = lo.astype(jnp.float32)

        zf = jnp.zeros((8, _SCOLS), jnp.float32)
        s_gt, c_gt, s_eq, c_eq = zf, zf, zf, zf
        for j in range(_SROWS // 8):
            p = p16_ref[pl.ds(j * 8, 8), :].astype(jnp.float32)
            v = val_ref[pl.ds(j * 8, 8), :]
            gt = p > tf
            eq = p == tf
            s_gt = s_gt + jnp.where(gt, v, 0.0)
            c_gt = c_gt + jnp.where(gt, 1.0, 0.0)
            s_eq = s_eq + jnp.where(eq, v, 0.0)
            c_eq = c_eq + jnp.where(eq, 1.0, 0.0)
        sum_gt = jnp.sum(s_gt)
        cnt_gt = jnp.sum(c_gt)
        sum_eq = jnp.sum(s_eq)
        cnt_eq = jnp.maximum(jnp.sum(c_eq), 1.0)
        kth = sum_eq / cnt_eq
        out_ref[0, 0] = (sum_gt + (kf - cnt_gt) * kth) / kf


def kernel(logits, labels):
    out = pl.pallas_call(
        _ce_topk_kernel,
        grid=(_GB, _GH),
        in_specs=[
            pl.BlockSpec((1, _C, _HB, _W), lambda b, h: (b, 0, h, 0)),
            pl.BlockSpec((1, _HB, _W), lambda b, h: (b, h, 0)),
        ],
        out_specs=pl.BlockSpec(memory_space=pltpu.SMEM),
        out_shape=jax.ShapeDtypeStruct((1, 1), jnp.float32),
        scratch_shapes=[
            pltpu.VMEM((_SROWS, _SCOLS), jnp.float32),
            pltpu.VMEM((_SROWS, _SCOLS), jnp.int16),
        ],
    )(logits, labels)
    return out[0, 0]


# speedup vs baseline: 36.5493x; 1.0094x over previous
import jax
import jax.numpy as jnp
from jax.experimental import pallas as pl
from jax.experimental.pallas import tpu as pltpu

_IGNORE = 255
_B, _C, _H, _W = 8, 19, 512, 512
_HB = 256
_N = _B * _H * _W
_K = int(0.2 * _N)
_GB, _GH = _B, _H // _HB
_SROWS, _SCOLS = _H, _B * _W


def _ce_topk_kernel(logits_ref, labels_ref, out_ref, val_ref, p16_ref):
    b = pl.program_id(0)
    h = pl.program_id(1)

    x = logits_ref[0]
    lab = labels_ref[0]

    m = jnp.max(x, axis=0)
    s = jnp.sum(jnp.exp(x - m[None, :, :]), axis=0)
    cls = jax.lax.broadcasted_iota(jnp.int32, (_C, _HB, _W), 0)
    picked = jnp.sum(jnp.where(cls == lab[None, :, :], x, 0.0), axis=0)
    nll = (m - picked) + jnp.log(s)
    loss = jnp.where(lab != _IGNORE, nll, 0.0)

    val_ref[pl.ds(h * _HB, _HB), pl.ds(b * _W, _W)] = loss
    p16_ref[pl.ds(h * _HB, _HB), pl.ds(b * _W, _W)] = pltpu.bitcast(
        loss.astype(jnp.bfloat16), jnp.int16)

    @pl.when((b == _GB - 1) & (h == _GH - 1))
    def _select():
        kf = jnp.float32(_K)
        one16 = jnp.int16(1)

        def count_gt(mid):
            mid16 = mid.astype(jnp.int16)
            acc = jnp.zeros((16, _SCOLS), jnp.int16)
            for j in range(_SROWS // 16):
                blk = p16_ref[pl.ds(j * 16, 16), :]
                acc = acc + jnp.where(blk > mid16, one16, jnp.int16(0))
            return jnp.sum(acc.astype(jnp.float32))

        def body(_, carry):
            lo, hi = carry
            mid = lo + (hi - lo) // 2
            cnt = count_gt(mid)
            active = lo < hi
            below = cnt < kf
            new_hi = jnp.where(active & below, mid, hi)
            new_lo = jnp.where(active & (~below), mid + 1, lo)
            return new_lo, new_hi

        lo, _hi = jax.lax.fori_loop(
            0, 15, body, (jnp.int32(0), jnp.int32(0x7F80)))
        t16 = lo.astype(jnp.int16)

        zf = jnp.zeros((8, _SCOLS), jnp.float32)
        zi = jnp.zeros((8, _SCOLS), jnp.int16)
        s_gt, s_eq = zf, zf
        c_gt16, c_eq16 = zi, zi
        for j in range(_SROWS // 8):
            blk = p16_ref[pl.ds(j * 8, 8), :]
            v = val_ref[pl.ds(j * 8, 8), :]
            gt_i = jnp.where(blk > t16, one16, jnp.int16(0))
            eq_i = jnp.where(blk == t16, one16, jnp.int16(0))
            c_gt16 = c_gt16 + gt_i
            c_eq16 = c_eq16 + eq_i
            s_gt = s_gt + gt_i.astype(jnp.float32) * v
            s_eq = s_eq + eq_i.astype(jnp.float32) * v
        sum_gt = jnp.sum(s_gt)
        cnt_gt = jnp.sum(c_gt16.astype(jnp.float32))
        sum_eq = jnp.sum(s_eq)
        cnt_eq = jnp.maximum(jnp.sum(c_eq16.astype(jnp.float32)), 1.0)
        kth = sum_eq / cnt_eq
        out_ref[0, 0] = (sum_gt + (kf - cnt_gt) * kth) / kf


def kernel(logits, labels):
    out = pl.pallas_call(
        _ce_topk_kernel,
        grid=(_GB, _GH),
        in_specs=[
            pl.BlockSpec((1, _C, _HB, _W), lambda b, h: (b, 0, h, 0)),
            pl.BlockSpec((1, _HB, _W), lambda b, h: (b, h, 0)),
        ],
        out_specs=pl.BlockSpec(memory_space=pltpu.SMEM),
        out_shape=jax.ShapeDtypeStruct((1, 1), jnp.float32),
        scratch_shapes=[
            pltpu.VMEM((_SROWS, _SCOLS), jnp.float32),
            pltpu.VMEM((_SROWS, _SCOLS), jnp.int16),
        ],
    )(logits, labels)
    return out[0, 0]


# speedup vs baseline: 37.8224x; 1.0348x over previous
import jax
import jax.numpy as jnp
from jax.experimental import pallas as pl
from jax.experimental.pallas import tpu as pltpu

_IGNORE = 255
_B, _C, _H, _W = 8, 19, 512, 512
_HB = 256
_N = _B * _H * _W
_K = int(0.2 * _N)
_GB, _GH = _B, _H // _HB
_SROWS, _SCOLS = _H, _B * _W


def _ce_topk_kernel(logits_ref, labels_ref, out_ref, val_ref, p16_ref):
    b = pl.program_id(0)
    h = pl.program_id(1)

    lab = labels_ref[0]

    m = logits_ref[0, 0]
    for c in range(1, _C):
        m = jnp.maximum(m, logits_ref[0, c])
    s = jnp.zeros((_HB, _W), jnp.float32)
    picked = jnp.zeros((_HB, _W), jnp.float32)
    for c in range(_C):
        xc = logits_ref[0, c]
        s = s + jnp.exp(xc - m)
        picked = jnp.where(lab == c, xc, picked)
    nll = (m - picked) + jnp.log(s)
    loss = jnp.where(lab != _IGNORE, nll, 0.0)

    val_ref[pl.ds(h * _HB, _HB), pl.ds(b * _W, _W)] = loss
    p16_ref[pl.ds(h * _HB, _HB), pl.ds(b * _W, _W)] = pltpu.bitcast(
        loss.astype(jnp.bfloat16), jnp.int16)

    @pl.when((b == _GB - 1) & (h == _GH - 1))
    def _select():
        kf = jnp.float32(_K)
        one16 = jnp.int16(1)

        def count_gt(mid):
            mid16 = mid.astype(jnp.int16)
            acc = jnp.zeros((16, _SCOLS), jnp.int16)
            for j in range(_SROWS // 16):
                blk = p16_ref[pl.ds(j * 16, 16), :]
                acc = acc + jnp.where(blk > mid16, one16, jnp.int16(0))
            return jnp.sum(acc.astype(jnp.float32))

        def body(_, carry):
            lo, hi = carry
            mid = lo + (hi - lo) // 2
            cnt = count_gt(mid)
            active = lo < hi
            below = cnt < kf
            new_hi = jnp.where(active & below, mid, hi)
            new_lo = jnp.where(active & (~below), mid + 1, lo)
            return new_lo, new_hi

        lo, _hi = jax.lax.fori_loop(
            0, 15, body, (jnp.int32(0), jnp.int32(0x7F80)))
        t16 = lo.astype(jnp.int16)

        zf = jnp.zeros((8, _SCOLS), jnp.float32)
        zi = jnp.zeros((8, _SCOLS), jnp.int16)
        s_gt, s_eq = zf, zf
        c_gt16, c_eq16 = zi, zi
        for j in range(_SROWS // 8):
            blk = p16_ref[pl.ds(j * 8, 8), :]
            v = val_ref[pl.ds(j * 8, 8), :]
            gt_i = jnp.where(blk > t16, one16, jnp.int16(0))
            eq_i = jnp.where(blk == t16, one16, jnp.int16(0))
            c_gt16 = c_gt16 + gt_i
            c_eq16 = c_eq16 + eq_i
            s_gt = s_gt + gt_i.astype(jnp.float32) * v
            s_eq = s_eq + eq_i.astype(jnp.float32) * v
        sum_gt = jnp.sum(s_gt)
        cnt_gt = jnp.sum(c_gt16.astype(jnp.float32))
        sum_eq = jnp.sum(s_eq)
        cnt_eq = jnp.maximum(jnp.sum(c_eq16.astype(jnp.float32)), 1.0)
        kth = sum_eq / cnt_eq
        out_ref[0, 0] = (sum_gt + (kf - cnt_gt) * kth) / kf


def kernel(logits, labels):
    out = pl.pallas_call(
        _ce_topk_kernel,
        grid=(_GB, _GH),
        in_specs=[
            pl.BlockSpec((1, _C, _HB, _W), lambda b, h: (b, 0, h, 0)),
            pl.BlockSpec((1, _HB, _W), lambda b, h: (b, h, 0)),
        ],
        out_specs=pl.BlockSpec(memory_space=pltpu.SMEM),
        out_shape=jax.ShapeDtypeStruct((1, 1), jnp.float32),
        scratch_shapes=[
            pltpu.VMEM((_SROWS, _SCOLS), jnp.float32),
            pltpu.VMEM((_SROWS, _SCOLS), jnp.int16),
        ],
    )(logits, labels)
    return out[0, 0]
